# baseline (speedup 1.0000x reference)
import functools

import jax
import jax.numpy as jnp
from jax import lax
from jax.experimental import pallas as pl
from jax.experimental.pallas import tpu as pltpu
from jax.experimental.pallas import tpu_sc as plsc

D = 1024
L = 16
FB = D // L
T = 16


def _body(ids_hbm, pids_hbm, tts_hbm, wtab_hbm, ptab_hbm, ttab_hbm, out_hbm,
          widx_v, pidx_v, tt_v, acc0_v, acc1_v, acc2_v,
          prow0_v, prow1_v, prow2_v,
          ttab_v, diff_v, smem_s, sw0, sw1, sw2, sp0, sp1, sp2, so0, so1, so2,
          ntok_per_w, nchunks):
    nc = 2
    wid = lax.axis_index("s") * nc + lax.axis_index("c")
    seq = ids_hbm.shape[1]
    w_per_row = seq // ntok_per_w
    row = wid // w_per_row
    col = (wid % w_per_row) * ntok_per_w

    c0 = pltpu.async_copy(ids_hbm.at[row, pl.ds(col, ntok_per_w)], widx_v, sw0)
    c1 = pltpu.async_copy(pids_hbm.at[row, pl.ds(col, ntok_per_w)], pidx_v, sp0)
    c2 = pltpu.async_copy(tts_hbm.at[row, pl.ds(col, ntok_per_w)],
                          tt_v.at[pl.ds(0, ntok_per_w)], so0)
    c3 = pltpu.async_copy(ttab_hbm, ttab_v, sw1)
    c0.wait()
    c1.wait()
    c2.wait()
    c3.wait()
    for f in range(FB):
        sl = pl.ds(f * L, L)
        diff_v[sl] = ttab_v[1, sl] - ttab_v[0, sl]

    accs = [acc0_v, acc1_v, acc2_v]
    prows = [prow0_v, prow1_v, prow2_v]
    semw = [sw0, sw1, sw2]
    semp = [sp0, sp1, sp2]
    semo = [so0, so1, so2]
    nbuf = 3

    def word_desc(c):
        b = c % nbuf
        return pltpu.make_async_copy(
            wtab_hbm.at[widx_v.at[pl.ds(c * T, T)]], accs[b], semw[b])

    def pos_desc(c):
        b = c % nbuf
        return pltpu.make_async_copy(
            ptab_hbm.at[pidx_v.at[pl.ds(c * T, T)]], prows[b], semp[b])

    def out_desc(c):
        b = c % nbuf
        return pltpu.make_async_copy(
            accs[b], out_hbm.at[row].at[pl.ds(col + c * T, T)], semo[b])

    def compute(c):
        b = c % nbuf
        acc = accs[b]
        prow = prows[b]

        for j in range(T):
            smem_s[j] = tt_v[pl.ds(c * T + j, L)][0].astype(jnp.float32)

        @plsc.parallel_loop(0, FB, unroll=2)
        def fblk(f):
            sl = pl.ds(f * L, L)
            ttv = ttab_v[0, sl]
            dfv = diff_v[sl]
            for t in range(T):
                plsc.addupdate(acc.at[t, sl],
                               prow[t, sl] + ttv + smem_s[t] * dfv)

    for c in range(min(nbuf - 1, nchunks)):
        word_desc(c).start()
        pos_desc(c).start()

    for c in range(nchunks):
        p = c + nbuf - 1
        if p < nchunks:
            if p >= nbuf:
                out_desc(p - nbuf).wait()
            word_desc(p).start()
            pos_desc(p).start()
        word_desc(c).wait()
        pos_desc(c).wait()
        compute(c)
        out_desc(c).start()
    for c in range(max(nchunks - nbuf, 0), nchunks):
        out_desc(c).wait()


def kernel(input_ids, position_ids, token_type_ids, word_embeddings,
           position_embeddings, token_type_embeddings):
    b, s = input_ids.shape
    ntok = b * s
    d = word_embeddings.shape[1]
    info = plsc.get_sparse_core_info()
    nw = info.num_cores * info.num_subcores
    ntok_per_w = ntok // nw
    nchunks = ntok_per_w // T

    mesh = plsc.VectorSubcoreMesh(core_axis_name="c", subcore_axis_name="s")
    body = functools.partial(_body, ntok_per_w=ntok_per_w, nchunks=nchunks)
    fn = pl.kernel(
        body,
        mesh=mesh,
        out_type=jax.ShapeDtypeStruct((b, s, d), jnp.float32),
        scratch_types=[
            pltpu.VMEM((ntok_per_w,), jnp.int32),
            pltpu.VMEM((ntok_per_w,), jnp.int32),
            pltpu.VMEM((ntok_per_w + L,), jnp.int32),
            pltpu.VMEM((T, D), jnp.float32),
            pltpu.VMEM((T, D), jnp.float32),
            pltpu.VMEM((T, D), jnp.float32),
            pltpu.VMEM((T, D), jnp.float32),
            pltpu.VMEM((T, D), jnp.float32),
            pltpu.VMEM((T, D), jnp.float32),
            pltpu.VMEM((2, D), jnp.float32),
            pltpu.VMEM((D,), jnp.float32),
            pltpu.SMEM((T,), jnp.float32),
            pltpu.SemaphoreType.DMA,
            pltpu.SemaphoreType.DMA,
            pltpu.SemaphoreType.DMA,
            pltpu.SemaphoreType.DMA,
            pltpu.SemaphoreType.DMA,
            pltpu.SemaphoreType.DMA,
            pltpu.SemaphoreType.DMA,
            pltpu.SemaphoreType.DMA,
            pltpu.SemaphoreType.DMA,
        ],
    )
    return fn(input_ids, position_ids, token_type_ids, word_embeddings,
              position_embeddings, token_type_embeddings)


# speedup vs baseline: 1.0674x; 1.0674x over previous
import functools

import jax
import jax.numpy as jnp
from jax import lax
from jax.experimental import pallas as pl
from jax.experimental.pallas import tpu as pltpu
from jax.experimental.pallas import tpu_sc as plsc

D = 1024
L = 16
FB = D // L
T = 16


def _body(ids_hbm, pids_hbm, tts_hbm, wtab_hbm, ptab_hbm, ttab_hbm, out_hbm,
          widx_v, pidx_v, tt_v, acc0_v, acc1_v, prow0_v, prow1_v,
          ttab_v, diff_v, smem_s, sw0, sw1, sp0, sp1, so0, so1,
          ntok_per_w, nchunks):
    nc = 2
    wid = lax.axis_index("s") * nc + lax.axis_index("c")
    seq = ids_hbm.shape[1]
    w_per_row = seq // ntok_per_w
    row = wid // w_per_row
    col = (wid % w_per_row) * ntok_per_w

    c0 = pltpu.async_copy(ids_hbm.at[row, pl.ds(col, ntok_per_w)], widx_v, sw0)
    c1 = pltpu.async_copy(pids_hbm.at[row, pl.ds(col, ntok_per_w)], pidx_v, sp0)
    c2 = pltpu.async_copy(tts_hbm.at[row, pl.ds(col, ntok_per_w)],
                          tt_v.at[pl.ds(0, ntok_per_w)], so0)
    c3 = pltpu.async_copy(ttab_hbm, ttab_v, sw1)
    c0.wait()
    c1.wait()
    c2.wait()
    c3.wait()
    for f in range(FB):
        sl = pl.ds(f * L, L)
        diff_v[sl] = ttab_v[1, sl] - ttab_v[0, sl]

    accs = [acc0_v, acc1_v]
    prows = [prow0_v, prow1_v]
    semw = [sw0, sw1]
    semp = [sp0, sp1]
    semo = [so0, so1]

    def word_desc(c, b):
        return pltpu.make_async_copy(
            wtab_hbm.at[widx_v.at[pl.ds(c * T, T)]], accs[b], semw[b])

    def pos_desc(c, b):
        return pltpu.make_async_copy(
            ptab_hbm.at[pidx_v.at[pl.ds(c * T, T)]], prows[b], semp[b])

    def out_desc(c, b):
        return pltpu.make_async_copy(
            accs[b], out_hbm.at[row].at[pl.ds(col + c * T, T)], semo[b])

    def compute(c, b):
        acc = accs[b]
        prow = prows[b]

        for j in range(T):
            smem_s[j] = tt_v[pl.ds(c * T + j, L)][0].astype(jnp.float32)

        @plsc.parallel_loop(0, FB, unroll=4)
        def fblk(f):
            sl = pl.ds(f * L, L)
            ttv = ttab_v[0, sl]
            dfv = diff_v[sl]
            for t in range(T):
                plsc.addupdate(acc.at[t, sl],
                               prow[t, sl] + ttv + smem_s[t] * dfv)

    word_desc(0, 0).start()
    pos_desc(0, 0).start()

    def g_body(g, _):
        for b in range(2):
            c = 2 * g + b
            nb = 1 - b

            @pl.when(c >= 1)
            def _():
                out_desc(c - 1, nb).wait()

            @pl.when(c + 1 < nchunks)
            def _():
                word_desc(c + 1, nb).start()
                pos_desc(c + 1, nb).start()

            word_desc(c, b).wait()
            pos_desc(c, b).wait()
            compute(c, b)
            out_desc(c, b).start()
        return 0

    lax.fori_loop(0, nchunks // 2, g_body, 0)
    out_desc(nchunks - 1, (nchunks - 1) % 2).wait()


def kernel(input_ids, position_ids, token_type_ids, word_embeddings,
           position_embeddings, token_type_embeddings):
    b, s = input_ids.shape
    ntok = b * s
    d = word_embeddings.shape[1]
    info = plsc.get_sparse_core_info()
    nw = info.num_cores * info.num_subcores
    ntok_per_w = ntok // nw
    nchunks = ntok_per_w // T

    mesh = plsc.VectorSubcoreMesh(core_axis_name="c", subcore_axis_name="s")
    body = functools.partial(_body, ntok_per_w=ntok_per_w, nchunks=nchunks)
    fn = pl.kernel(
        body,
        mesh=mesh,
        out_type=jax.ShapeDtypeStruct((b, s, d), jnp.float32),
        scratch_types=[
            pltpu.VMEM((ntok_per_w,), jnp.int32),
            pltpu.VMEM((ntok_per_w,), jnp.int32),
            pltpu.VMEM((ntok_per_w + L,), jnp.int32),
            pltpu.VMEM((T, D), jnp.float32),
            pltpu.VMEM((T, D), jnp.float32),
            pltpu.VMEM((T, D), jnp.float32),
            pltpu.VMEM((T, D), jnp.float32),
            pltpu.VMEM((2, D), jnp.float32),
            pltpu.VMEM((D,), jnp.float32),
            pltpu.SMEM((T,), jnp.float32),
            pltpu.SemaphoreType.DMA,
            pltpu.SemaphoreType.DMA,
            pltpu.SemaphoreType.DMA,
            pltpu.SemaphoreType.DMA,
            pltpu.SemaphoreType.DMA,
            pltpu.SemaphoreType.DMA,
        ],
    )
    return fn(input_ids, position_ids, token_type_ids, word_embeddings,
              position_embeddings, token_type_embeddings)


# speedup vs baseline: 1.1013x; 1.0317x over previous
import functools

import jax
import jax.numpy as jnp
from jax import lax
from jax.experimental import pallas as pl
from jax.experimental.pallas import tpu as pltpu
from jax.experimental.pallas import tpu_sc as plsc

D = 1024
L = 16
FB = D // L
T = 16


def _body(ids_hbm, pids_hbm, tts_hbm, wtab_hbm, ptab_hbm, ttab_hbm, out_hbm,
          widx_v, pidx_v, tt_v, acc0_v, acc1_v, prow0_v, prow1_v,
          ttab_v, diff_v, smem_s, sw0, sw1, sp0, sp1, so0, so1,
          ntok_per_w, nchunks):
    nc = 2
    wid = lax.axis_index("s") * nc + lax.axis_index("c")
    seq = ids_hbm.shape[1]
    w_per_row = seq // ntok_per_w
    row = wid // w_per_row
    col = (wid % w_per_row) * ntok_per_w

    c0 = pltpu.async_copy(ids_hbm.at[row, pl.ds(col, ntok_per_w)], widx_v, sw0)
    c1 = pltpu.async_copy(pids_hbm.at[row, pl.ds(col, ntok_per_w)], pidx_v, sp0)
    c2 = pltpu.async_copy(tts_hbm.at[row, pl.ds(col, ntok_per_w)],
                          tt_v.at[pl.ds(0, ntok_per_w)], so0)
    c3 = pltpu.async_copy(ttab_hbm, ttab_v, sw1)
    c0.wait()
    c1.wait()
    c2.wait()
    c3.wait()
    for f in range(FB):
        sl = pl.ds(f * L, L)
        diff_v[sl] = ttab_v[1, sl] - ttab_v[0, sl]

    accs = [acc0_v, acc1_v]
    prows = [prow0_v, prow1_v]
    semw = [sw0, sw1]
    semp = [sp0, sp1]
    semo = [so0, so1]

    def word_desc(c, b):
        return pltpu.make_async_copy(
            wtab_hbm.at[widx_v.at[pl.ds(c * T, T)]], accs[b], semw[b])

    def pos_desc(c, b):
        return pltpu.make_async_copy(
            ptab_hbm.at[pidx_v.at[pl.ds(c * T, T)]], prows[b], semp[b])

    def out_desc(c, b):
        return pltpu.make_async_copy(
            accs[b], out_hbm.at[row].at[pl.ds(col + c * T, T)], semo[b])

    def compute(c, b):
        acc = accs[b]
        prow = prows[b]

        for j in range(T):
            smem_s[j] = tt_v[pl.ds(c * T + j, L)][0].astype(jnp.float32)

        @plsc.parallel_loop(0, FB, unroll=2)
        def fblk(f):
            sl = pl.ds(f * L, L)
            ttv = ttab_v[0, sl]
            dfv = diff_v[sl]
            for t in range(T):
                plsc.addupdate(acc.at[t, sl],
                               prow[t, sl] + ttv + smem_s[t] * dfv)

    word_desc(0, 0).start()
    pos_desc(0, 0).start()

    def g_body(g, _):
        for b in range(2):
            c = 2 * g + b
            nb = 1 - b

            @pl.when(c + 1 < nchunks)
            def _():
                pos_desc(c + 1, nb).start()

            @pl.when(c >= 1)
            def _():
                out_desc(c - 1, nb).wait()

            @pl.when(c + 1 < nchunks)
            def _():
                word_desc(c + 1, nb).start()

            word_desc(c, b).wait()
            pos_desc(c, b).wait()
            compute(c, b)
            out_desc(c, b).start()
        return 0

    lax.fori_loop(0, nchunks // 2, g_body, 0)
    out_desc(nchunks - 1, (nchunks - 1) % 2).wait()


def kernel(input_ids, position_ids, token_type_ids, word_embeddings,
           position_embeddings, token_type_embeddings):
    b, s = input_ids.shape
    ntok = b * s
    d = word_embeddings.shape[1]
    info = plsc.get_sparse_core_info()
    nw = info.num_cores * info.num_subcores
    ntok_per_w = ntok // nw
    nchunks = ntok_per_w // T

    mesh = plsc.VectorSubcoreMesh(core_axis_name="c", subcore_axis_name="s")
    body = functools.partial(_body, ntok_per_w=ntok_per_w, nchunks=nchunks)
    fn = pl.kernel(
        body,
        mesh=mesh,
        out_type=jax.ShapeDtypeStruct((b, s, d), jnp.float32),
        scratch_types=[
            pltpu.VMEM((ntok_per_w,), jnp.int32),
            pltpu.VMEM((ntok_per_w,), jnp.int32),
            pltpu.VMEM((ntok_per_w + L,), jnp.int32),
            pltpu.VMEM((T, D), jnp.float32),
            pltpu.VMEM((T, D), jnp.float32),
            pltpu.VMEM((T, D), jnp.float32),
            pltpu.VMEM((T, D), jnp.float32),
            pltpu.VMEM((2, D), jnp.float32),
            pltpu.VMEM((D,), jnp.float32),
            pltpu.SMEM((T,), jnp.float32),
            pltpu.SemaphoreType.DMA,
            pltpu.SemaphoreType.DMA,
            pltpu.SemaphoreType.DMA,
            pltpu.SemaphoreType.DMA,
            pltpu.SemaphoreType.DMA,
            pltpu.SemaphoreType.DMA,
        ],
    )
    return fn(input_ids, position_ids, token_type_ids, word_embeddings,
              position_embeddings, token_type_embeddings)


# speedup vs baseline: 1.1302x; 1.0262x over previous
import functools

import jax
import jax.numpy as jnp
from jax import lax
from jax.experimental import pallas as pl
from jax.experimental.pallas import tpu as pltpu
from jax.experimental.pallas import tpu_sc as plsc

D = 1024
L = 16
FB = D // L
T = 16


def _body(ids_hbm, pids_hbm, tts_hbm, wtab_hbm, ptab_hbm, ttab_hbm, out_hbm,
          widx_v, pidx_v, tt_v, acc0_v, acc1_v, prow0_v, prow1_v,
          ttab_v, diff_v, smem_s, sw0, sw1, sp0, sp1, so0, so1,
          ntok_per_w, nchunks):
    nc = 2
    wid = lax.axis_index("s") * nc + lax.axis_index("c")
    seq = ids_hbm.shape[1]
    w_per_row = seq // ntok_per_w
    row = wid // w_per_row
    col = (wid % w_per_row) * ntok_per_w

    c0 = pltpu.async_copy(ids_hbm.at[row, pl.ds(col, ntok_per_w)], widx_v, sw0)
    c1 = pltpu.async_copy(pids_hbm.at[row, pl.ds(col, ntok_per_w)], pidx_v, sp0)
    c2 = pltpu.async_copy(tts_hbm.at[row, pl.ds(col, ntok_per_w)],
                          tt_v.at[pl.ds(0, ntok_per_w)], so0)
    c3 = pltpu.async_copy(ttab_hbm, ttab_v, sw1)

    accs = [acc0_v, acc1_v]
    prows = [prow0_v, prow1_v]
    semw = [sw0, sw1]
    semp = [sp0, sp1]
    semo = [so0, so1]

    def word_desc(c, b):
        return pltpu.make_async_copy(
            wtab_hbm.at[widx_v.at[pl.ds(c * T, T)]], accs[b], semw[b])

    def pos_desc(c, b):
        return pltpu.make_async_copy(
            ptab_hbm.at[pidx_v.at[pl.ds(c * T, T)]], prows[b], semp[b])

    def out_desc(c, b):
        return pltpu.make_async_copy(
            accs[b], out_hbm.at[row].at[pl.ds(col + c * T, T)], semo[b])

    def compute(c, b):
        acc = accs[b]
        prow = prows[b]

        for j in range(T):
            smem_s[j] = tt_v[pl.ds(c * T + j, L)][0].astype(jnp.float32)

        @plsc.parallel_loop(0, FB, unroll=2)
        def fblk(f):
            sl = pl.ds(f * L, L)
            ttv = ttab_v[0, sl]
            dfv = diff_v[sl]
            for t in range(T):
                plsc.addupdate(acc.at[t, sl],
                               prow[t, sl] + ttv + smem_s[t] * dfv)

    c0.wait()
    word_desc(0, 0).start()
    c1.wait()
    pos_desc(0, 0).start()
    c2.wait()
    c3.wait()
    for f in range(FB):
        sl = pl.ds(f * L, L)
        diff_v[sl] = ttab_v[1, sl] - ttab_v[0, sl]

    def g_body(g, _):
        for b in range(2):
            c = 2 * g + b
            nb = 1 - b

            @pl.when(c + 1 < nchunks)
            def _():
                pos_desc(c + 1, nb).start()

            @pl.when(c >= 1)
            def _():
                out_desc(c - 1, nb).wait()

            @pl.when(c + 1 < nchunks)
            def _():
                word_desc(c + 1, nb).start()

            word_desc(c, b).wait()
            pos_desc(c, b).wait()
            compute(c, b)
            out_desc(c, b).start()
        return 0

    lax.fori_loop(0, nchunks // 2, g_body, 0)
    out_desc(nchunks - 1, (nchunks - 1) % 2).wait()


def kernel(input_ids, position_ids, token_type_ids, word_embeddings,
           position_embeddings, token_type_embeddings):
    b, s = input_ids.shape
    ntok = b * s
    d = word_embeddings.shape[1]
    info = plsc.get_sparse_core_info()
    nw = info.num_cores * info.num_subcores
    ntok_per_w = ntok // nw
    nchunks = ntok_per_w // T

    mesh = plsc.VectorSubcoreMesh(core_axis_name="c", subcore_axis_name="s")
    body = functools.partial(_body, ntok_per_w=ntok_per_w, nchunks=nchunks)
    fn = pl.kernel(
        body,
        mesh=mesh,
        out_type=jax.ShapeDtypeStruct((b, s, d), jnp.float32),
        scratch_types=[
            pltpu.VMEM((ntok_per_w,), jnp.int32),
            pltpu.VMEM((ntok_per_w,), jnp.int32),
            pltpu.VMEM((ntok_per_w + L,), jnp.int32),
            pltpu.VMEM((T, D), jnp.float32),
            pltpu.VMEM((T, D), jnp.float32),
            pltpu.VMEM((T, D), jnp.float32),
            pltpu.VMEM((T, D), jnp.float32),
            pltpu.VMEM((2, D), jnp.float32),
            pltpu.VMEM((D,), jnp.float32),
            pltpu.SMEM((T,), jnp.float32),
            pltpu.SemaphoreType.DMA,
            pltpu.SemaphoreType.DMA,
            pltpu.SemaphoreType.DMA,
            pltpu.SemaphoreType.DMA,
            pltpu.SemaphoreType.DMA,
            pltpu.SemaphoreType.DMA,
        ],
    )
    return fn(input_ids, position_ids, token_type_ids, word_embeddings,
              position_embeddings, token_type_embeddings)


# speedup vs baseline: 1.1381x; 1.0070x over previous
import functools

import jax
import jax.numpy as jnp
from jax import lax
from jax.experimental import pallas as pl
from jax.experimental.pallas import tpu as pltpu
from jax.experimental.pallas import tpu_sc as plsc

D = 1024
L = 16
FB = D // L
T = 16


def _body(ids_hbm, pids_hbm, tts_hbm, wtab_hbm, ptab_hbm, ttab_hbm, out_hbm,
          widx_v, pidx_v, tt_v, acc0_v, acc1_v, prow0_v, prow1_v,
          ttab_v, diff_v, smem_s, sw0, sw1, sp0, sp1, so0, so1,
          ntok_per_w, nchunks):
    nc = 2
    wid = lax.axis_index("s") * nc + lax.axis_index("c")
    seq = ids_hbm.shape[1]
    w_per_row = seq // ntok_per_w
    row = wid // w_per_row
    col = (wid % w_per_row) * ntok_per_w

    c0 = pltpu.async_copy(ids_hbm.at[row, pl.ds(col, ntok_per_w)], widx_v, sw0)
    c1 = pltpu.async_copy(pids_hbm.at[row, pl.ds(col, ntok_per_w)], pidx_v, sp0)
    c2 = pltpu.async_copy(tts_hbm.at[row, pl.ds(col, ntok_per_w)],
                          tt_v.at[pl.ds(0, ntok_per_w)], so0)
    c3 = pltpu.async_copy(ttab_hbm, ttab_v, sw1)

    accs = [acc0_v, acc1_v]
    prows = [prow0_v, prow1_v]
    semw = [sw0, sw1]
    semp = [sp0, sp1]
    semo = [so0, so1]

    def word_desc(c, b):
        return pltpu.make_async_copy(
            wtab_hbm.at[widx_v.at[pl.ds(c * T, T)]], accs[b], semw[b])

    def pos_desc(c, b):
        return pltpu.make_async_copy(
            ptab_hbm.at[pidx_v.at[pl.ds(c * T, T)]], prows[b], semp[b])

    def out_desc(c, b):
        return pltpu.make_async_copy(
            accs[b], out_hbm.at[row].at[pl.ds(col + c * T, T)], semo[b])

    def compute(c, b):
        acc = accs[b]
        prow = prows[b]

        def stage_tt(j, _):
            smem_s[j] = tt_v[pl.ds(c * T + j, L)][0].astype(jnp.float32)
            return 0

        lax.fori_loop(0, T, stage_tt, 0)

        @plsc.parallel_loop(0, FB, unroll=2)
        def fblk(f):
            sl = pl.ds(f * L, L)
            ttv = ttab_v[0, sl]
            dfv = diff_v[sl]
            for t in range(T):
                plsc.addupdate(acc.at[t, sl],
                               prow[t, sl] + ttv + smem_s[t] * dfv)

    c0.wait()
    word_desc(0, 0).start()
    c1.wait()
    pos_desc(0, 0).start()
    c2.wait()
    c3.wait()

    @plsc.parallel_loop(0, FB)
    def mkdiff(f):
        sl = pl.ds(f * L, L)
        diff_v[sl] = ttab_v[1, sl] - ttab_v[0, sl]

    def g_body(g, _):
        for b in range(2):
            c = 2 * g + b
            nb = 1 - b

            @pl.when(c + 1 < nchunks)
            def _():
                pos_desc(c + 1, nb).start()

            @pl.when(c >= 1)
            def _():
                out_desc(c - 1, nb).wait()

            @pl.when(c + 1 < nchunks)
            def _():
                word_desc(c + 1, nb).start()

            word_desc(c, b).wait()
            pos_desc(c, b).wait()
            compute(c, b)
            out_desc(c, b).start()
        return 0

    lax.fori_loop(0, nchunks // 2, g_body, 0)
    out_desc(nchunks - 1, (nchunks - 1) % 2).wait()


def kernel(input_ids, position_ids, token_type_ids, word_embeddings,
           position_embeddings, token_type_embeddings):
    b, s = input_ids.shape
    ntok = b * s
    d = word_embeddings.shape[1]
    info = plsc.get_sparse_core_info()
    nw = info.num_cores * info.num_subcores
    ntok_per_w = ntok // nw
    nchunks = ntok_per_w // T

    mesh = plsc.VectorSubcoreMesh(core_axis_name="c", subcore_axis_name="s")
    body = functools.partial(_body, ntok_per_w=ntok_per_w, nchunks=nchunks)
    fn = pl.kernel(
        body,
        mesh=mesh,
        out_type=jax.ShapeDtypeStruct((b, s, d), jnp.float32),
        scratch_types=[
            pltpu.VMEM((ntok_per_w,), jnp.int32),
            pltpu.VMEM((ntok_per_w,), jnp.int32),
            pltpu.VMEM((ntok_per_w + L,), jnp.int32),
            pltpu.VMEM((T, D), jnp.float32),
            pltpu.VMEM((T, D), jnp.float32),
            pltpu.VMEM((T, D), jnp.float32),
            pltpu.VMEM((T, D), jnp.float32),
            pltpu.VMEM((2, D), jnp.float32),
            pltpu.VMEM((D,), jnp.float32),
            pltpu.SMEM((T,), jnp.float32),
            pltpu.SemaphoreType.DMA,
            pltpu.SemaphoreType.DMA,
            pltpu.SemaphoreType.DMA,
            pltpu.SemaphoreType.DMA,
            pltpu.SemaphoreType.DMA,
            pltpu.SemaphoreType.DMA,
        ],
    )
    return fn(input_ids, position_ids, token_type_ids, word_embeddings,
              position_embeddings, token_type_embeddings)
